# Initial kernel scaffold; baseline (speedup 1.0000x reference)
import jax
import jax.numpy as jnp
from jax.experimental import pallas as pl


def kernel(x, gr_in_w, gr_in_b, gr_out_w, gr_out_b, cn_w, cn_b, patterns, nn_in_w, nn_in_b, nn_out_w, nn_out_b, ln_g, ln_b, pw, po, k_input, k_process):
    raise NotImplementedError("write your pallas kernel here")


# speedup vs baseline: 2.6896x; 2.6896x over previous
import math

import jax
import jax.numpy as jnp
from jax.experimental import pallas as pl

S = 2048
D = 1024
NI = 64
NP = 128
NH = 16
HD = D // NH
NNH = 4
NHD = NI // NNH
KIN = 32
KPR = 64

BQ = 512
BR = 256


def _gelu(x):
    return 0.5 * x * (1.0 + jax.lax.erf(x * (1.0 / math.sqrt(2.0))))


def _dot_t(a, b):
    return jax.lax.dot_general(a, b, (((1,), (1,)), ((), ())),
                               preferred_element_type=jnp.float32)


def _qkv_body(x_ref, w_ref, b_ref, q_ref, k_ref, v_ref):
    y = _dot_t(x_ref[:], w_ref[:]) + b_ref[:]
    q_ref[:] = y[:, :D]
    k_ref[:] = y[:, D:2 * D]
    v_ref[:] = y[:, 2 * D:]


def _attn_body(q_ref, k_ref, v_ref, o_ref):
    for h in (0, 1):
        sl = slice(h * HD, (h + 1) * HD)
        s = _dot_t(q_ref[:, sl], k_ref[:, sl]) * (1.0 / math.sqrt(HD))
        m = jnp.max(s, axis=1, keepdims=True)
        p = jnp.exp(s - m)
        l = jnp.sum(p, axis=1, keepdims=True)
        o_ref[:, sl] = jnp.dot(p / l, v_ref[:, sl],
                               preferred_element_type=jnp.float32)


def _post_body(a_ref, wo_ref, bo_ref, cw_ref, cb_ref, pt_ref, nw_ref, nb_ref,
               loc_ref, qn_ref, kn_ref, vn_ref, nsum_ref, nmax_ref):
    i = pl.program_id(0)
    att = _dot_t(a_ref[:], wo_ref[:]) + bo_ref[:]
    na = _dot_t(att, cw_ref[:]) + cb_ref[:]
    loc = _gelu(_dot_t(att, pt_ref[:]))
    loc_ref[:] = loc
    qkvn = _dot_t(loc, nw_ref[:]) + nb_ref[:]
    qn_ref[:] = qkvn[:, :NI]
    kn_ref[:] = qkvn[:, NI:2 * NI]
    vn_ref[:] = qkvn[:, 2 * NI:]
    psum = jnp.sum(na, axis=0, keepdims=True)
    pmax = jnp.max(na, axis=0, keepdims=True)

    @pl.when(i == 0)
    def _():
        nsum_ref[:] = psum
        nmax_ref[:] = pmax

    @pl.when(i != 0)
    def _():
        nsum_ref[:] = nsum_ref[:] + psum
        nmax_ref[:] = jnp.maximum(nmax_ref[:], pmax)


def _route1_body(ns_ref, nm_ref, w_ref):
    mn = ns_ref[:] * (1.0 / S)
    mx = nm_ref[:]
    fs = 0.5 * mn + 0.3 * mx + 0.2 * mn
    p = fs - jnp.max(fs, axis=1, keepdims=True)
    e = jnp.exp(p)
    probs = e / jnp.sum(e, axis=1, keepdims=True)
    fb = jnp.broadcast_to(fs, (NI, NI))
    fa = fb.T
    il = jax.lax.broadcasted_iota(jnp.int32, (NI, NI), 0)
    jl = jax.lax.broadcasted_iota(jnp.int32, (NI, NI), 1)
    beats = (fa > fb) | ((fa == fb) & (il < jl))
    rank = jnp.sum(beats.astype(jnp.float32), axis=0, keepdims=True)
    sel = rank < float(KIN)
    w_ref[:] = jnp.where(sel, (1.0 - probs) + probs, 0.0)


def _nattn_body(qn_ref, kn_ref, vn_ref, loc_ref, g_ref, b_ref, ow_ref, ob_ref,
                acts_ref):
    qn = qn_ref[:]
    kn = kn_ref[:]
    vn = vn_ref[:]
    outs = []
    for h in range(NNH):
        sl = slice(h * NHD, (h + 1) * NHD)
        s = _dot_t(qn[:, sl], kn[:, sl]) * (1.0 / math.sqrt(NHD))
        m = jnp.max(s, axis=1, keepdims=True)
        p = jnp.exp(s - m)
        l = jnp.sum(p, axis=1, keepdims=True)
        outs.append(jnp.dot(p / l, vn[:, sl], preferred_element_type=jnp.float32))
    ao = _dot_t(jnp.concatenate(outs, axis=1), ow_ref[:]) + ob_ref[:]
    h_ = loc_ref[:] + ao
    mu = jnp.mean(h_, axis=1, keepdims=True)
    var = jnp.mean((h_ - mu) ** 2, axis=1, keepdims=True)
    acts_ref[:] = g_ref[:] * (h_ - mu) / jnp.sqrt(var + 1e-5) + b_ref[:]


def _pa_body(acts_ref, pw_ref, w_ref, pa_ref, ps_ref):
    i = pl.program_id(0)
    pa = _gelu(_dot_t(acts_ref[:], pw_ref[:] * w_ref[:]))
    pa_ref[:] = pa
    part = jnp.sum(pa, axis=0, keepdims=True)

    @pl.when(i == 0)
    def _():
        ps_ref[:] = part

    @pl.when(i != 0)
    def _():
        ps_ref[:] = ps_ref[:] + part


def _route2_body(ps_ref, m_ref):
    ps = ps_ref[:] * (1.0 / S)
    fb = jnp.broadcast_to(ps, (NP, NP))
    fa = fb.T
    il = jax.lax.broadcasted_iota(jnp.int32, (NP, NP), 0)
    jl = jax.lax.broadcasted_iota(jnp.int32, (NP, NP), 1)
    beats = (fa > fb) | ((fa == fb) & (il < jl))
    rank = jnp.sum(beats.astype(jnp.float32), axis=0, keepdims=True)
    m_ref[:] = (rank < float(KPR)).astype(jnp.float32)


def _out_body(pa_ref, m_ref, po_ref, o_ref):
    o_ref[:] = jnp.dot(pa_ref[:] * m_ref[:], po_ref[:],
                       preferred_element_type=jnp.float32)


def kernel(x, gr_in_w, gr_in_b, gr_out_w, gr_out_b, cn_w, cn_b, patterns,
           nn_in_w, nn_in_b, nn_out_w, nn_out_b, ln_g, ln_b, pw, po,
           k_input, k_process):
    f32 = jnp.float32
    x2 = x.reshape(S, D)

    q, k, v = pl.pallas_call(
        _qkv_body,
        grid=(S // BR,),
        in_specs=[
            pl.BlockSpec((BR, D), lambda i: (i, 0)),
            pl.BlockSpec((3 * D, D), lambda i: (0, 0)),
            pl.BlockSpec((1, 3 * D), lambda i: (0, 0)),
        ],
        out_specs=[pl.BlockSpec((BR, D), lambda i: (i, 0))] * 3,
        out_shape=[jax.ShapeDtypeStruct((S, D), f32)] * 3,
    )(x2, gr_in_w, gr_in_b.reshape(1, 3 * D))

    attn_out = pl.pallas_call(
        _attn_body,
        grid=(NH // 2, S // BQ),
        in_specs=[
            pl.BlockSpec((BQ, 2 * HD), lambda p, i: (i, p)),
            pl.BlockSpec((S, 2 * HD), lambda p, i: (0, p)),
            pl.BlockSpec((S, 2 * HD), lambda p, i: (0, p)),
        ],
        out_specs=pl.BlockSpec((BQ, 2 * HD), lambda p, i: (i, p)),
        out_shape=jax.ShapeDtypeStruct((S, D), f32),
    )(q, k, v)

    local, qn, kn, vn, nsum, nmax = pl.pallas_call(
        _post_body,
        grid=(S // BR,),
        in_specs=[
            pl.BlockSpec((BR, D), lambda i: (i, 0)),
            pl.BlockSpec((D, D), lambda i: (0, 0)),
            pl.BlockSpec((1, D), lambda i: (0, 0)),
            pl.BlockSpec((NI, D), lambda i: (0, 0)),
            pl.BlockSpec((1, NI), lambda i: (0, 0)),
            pl.BlockSpec((NI, D), lambda i: (0, 0)),
            pl.BlockSpec((3 * NI, NI), lambda i: (0, 0)),
            pl.BlockSpec((1, 3 * NI), lambda i: (0, 0)),
        ],
        out_specs=[
            pl.BlockSpec((BR, NI), lambda i: (i, 0)),
            pl.BlockSpec((BR, NI), lambda i: (i, 0)),
            pl.BlockSpec((BR, NI), lambda i: (i, 0)),
            pl.BlockSpec((BR, NI), lambda i: (i, 0)),
            pl.BlockSpec((1, NI), lambda i: (0, 0)),
            pl.BlockSpec((1, NI), lambda i: (0, 0)),
        ],
        out_shape=[
            jax.ShapeDtypeStruct((S, NI), f32),
            jax.ShapeDtypeStruct((S, NI), f32),
            jax.ShapeDtypeStruct((S, NI), f32),
            jax.ShapeDtypeStruct((S, NI), f32),
            jax.ShapeDtypeStruct((1, NI), f32),
            jax.ShapeDtypeStruct((1, NI), f32),
        ],
    )(attn_out, gr_out_w, gr_out_b.reshape(1, D), cn_w, cn_b.reshape(1, NI),
      patterns, nn_in_w, nn_in_b.reshape(1, 3 * NI))

    w = pl.pallas_call(
        _route1_body,
        in_specs=[pl.BlockSpec((1, NI), lambda: (0, 0)),
                  pl.BlockSpec((1, NI), lambda: (0, 0))],
        out_specs=pl.BlockSpec((1, NI), lambda: (0, 0)),
        out_shape=jax.ShapeDtypeStruct((1, NI), f32),
    )(nsum, nmax)

    acts = pl.pallas_call(
        _nattn_body,
        grid=(S // BQ,),
        in_specs=[
            pl.BlockSpec((BQ, NI), lambda i: (i, 0)),
            pl.BlockSpec((S, NI), lambda i: (0, 0)),
            pl.BlockSpec((S, NI), lambda i: (0, 0)),
            pl.BlockSpec((BQ, NI), lambda i: (i, 0)),
            pl.BlockSpec((1, NI), lambda i: (0, 0)),
            pl.BlockSpec((1, NI), lambda i: (0, 0)),
            pl.BlockSpec((NI, NI), lambda i: (0, 0)),
            pl.BlockSpec((1, NI), lambda i: (0, 0)),
        ],
        out_specs=pl.BlockSpec((BQ, NI), lambda i: (i, 0)),
        out_shape=jax.ShapeDtypeStruct((S, NI), f32),
    )(qn, kn, vn, local, ln_g.reshape(1, NI), ln_b.reshape(1, NI),
      nn_out_w, nn_out_b.reshape(1, NI))

    pa, ps = pl.pallas_call(
        _pa_body,
        grid=(S // BR,),
        in_specs=[
            pl.BlockSpec((BR, NI), lambda i: (i, 0)),
            pl.BlockSpec((NP, NI), lambda i: (0, 0)),
            pl.BlockSpec((1, NI), lambda i: (0, 0)),
        ],
        out_specs=[
            pl.BlockSpec((BR, NP), lambda i: (i, 0)),
            pl.BlockSpec((1, NP), lambda i: (0, 0)),
        ],
        out_shape=[
            jax.ShapeDtypeStruct((S, NP), f32),
            jax.ShapeDtypeStruct((1, NP), f32),
        ],
    )(acts, pw, w)

    mask2 = pl.pallas_call(
        _route2_body,
        in_specs=[pl.BlockSpec((1, NP), lambda: (0, 0))],
        out_specs=pl.BlockSpec((1, NP), lambda: (0, 0)),
        out_shape=jax.ShapeDtypeStruct((1, NP), f32),
    )(ps)

    out = pl.pallas_call(
        _out_body,
        grid=(S // BR,),
        in_specs=[
            pl.BlockSpec((BR, NP), lambda i: (i, 0)),
            pl.BlockSpec((1, NP), lambda i: (0, 0)),
            pl.BlockSpec((NP, D), lambda i: (0, 0)),
        ],
        out_specs=pl.BlockSpec((BR, D), lambda i: (i, 0)),
        out_shape=jax.ShapeDtypeStruct((S, D), f32),
    )(pa, mask2, po)

    return out.reshape(1, S, D)


# speedup vs baseline: 3.2468x; 1.2072x over previous
import math

import jax
import jax.numpy as jnp
from jax.experimental import pallas as pl

S = 2048
D = 1024
NI = 64
NP = 128
NH = 16
HD = D // NH
NNH = 4
NHD = NI // NNH
KIN = 32
KPR = 64

BQ = 512
BR = 256


def _gelu(x):
    return 0.5 * x * (1.0 + jax.lax.erf(x * (1.0 / math.sqrt(2.0))))


def _dot_t(a, b):
    return jax.lax.dot_general(a, b, (((1,), (1,)), ((), ())),
                               preferred_element_type=jnp.float32)


def _qkv_body(x_ref, w_ref, b_ref, q_ref, k_ref, v_ref):
    y = _dot_t(x_ref[:], w_ref[:]) + b_ref[:]
    q_ref[:] = y[:, :D]
    k_ref[:] = y[:, D:2 * D]
    v_ref[:] = y[:, 2 * D:]


def _attn_body(q_ref, k_ref, v_ref, o_ref):
    q = q_ref[:] * (1.0 / math.sqrt(HD))
    for h in (0, 1):
        sl = slice(h * HD, (h + 1) * HD)
        s = _dot_t(q[:, sl], k_ref[:, sl])
        m = jnp.max(s, axis=1, keepdims=True)
        p = jnp.exp(s - m)
        l = jnp.sum(p, axis=1, keepdims=True)
        o_ref[:, sl] = jnp.dot(p, v_ref[:, sl],
                               preferred_element_type=jnp.float32) / l


def _post_body(a_ref, wo_ref, bo_ref, cw_ref, cb_ref, pt_ref, nw_ref, nb_ref,
               loc_ref, qn_ref, kn_ref, vn_ref, nsum_ref, nmax_ref):
    i = pl.program_id(0)
    att = _dot_t(a_ref[:], wo_ref[:]) + bo_ref[:]
    na = _dot_t(att, cw_ref[:]) + cb_ref[:]
    loc = _gelu(_dot_t(att, pt_ref[:]))
    loc_ref[:] = loc
    qkvn = _dot_t(loc, nw_ref[:]) + nb_ref[:]
    qn_ref[:] = qkvn[:, :NI]
    kn_ref[:] = qkvn[:, NI:2 * NI]
    vn_ref[:] = qkvn[:, 2 * NI:]
    psum = jnp.sum(na, axis=0, keepdims=True)
    pmax = jnp.max(na, axis=0, keepdims=True)

    @pl.when(i == 0)
    def _():
        nsum_ref[:] = psum
        nmax_ref[:] = pmax

    @pl.when(i != 0)
    def _():
        nsum_ref[:] = nsum_ref[:] + psum
        nmax_ref[:] = jnp.maximum(nmax_ref[:], pmax)


def _route1_body(ns_ref, nm_ref, w_ref):
    mn = ns_ref[:] * (1.0 / S)
    mx = nm_ref[:]
    fs = 0.5 * mn + 0.3 * mx + 0.2 * mn
    p = fs - jnp.max(fs, axis=1, keepdims=True)
    e = jnp.exp(p)
    probs = e / jnp.sum(e, axis=1, keepdims=True)
    fb = jnp.broadcast_to(fs, (NI, NI))
    fa = fb.T
    il = jax.lax.broadcasted_iota(jnp.int32, (NI, NI), 0)
    jl = jax.lax.broadcasted_iota(jnp.int32, (NI, NI), 1)
    beats = (fa > fb) | ((fa == fb) & (il < jl))
    rank = jnp.sum(beats.astype(jnp.float32), axis=0, keepdims=True)
    sel = rank < float(KIN)
    w_ref[:] = jnp.where(sel, (1.0 - probs) + probs, 0.0)


def _nattn_body(qn_ref, kn_ref, vn_ref, loc_ref, g_ref, b_ref, ow_ref, ob_ref,
                acts_ref):
    qn = qn_ref[:] * (1.0 / math.sqrt(NHD))
    kn = kn_ref[:]
    vn = vn_ref[:]
    outs = []
    for h in range(NNH):
        sl = slice(h * NHD, (h + 1) * NHD)
        s = _dot_t(qn[:, sl], kn[:, sl])
        m = jnp.max(s, axis=1, keepdims=True)
        p = jnp.exp(s - m)
        l = jnp.sum(p, axis=1, keepdims=True)
        outs.append(jnp.dot(p, vn[:, sl],
                            preferred_element_type=jnp.float32) / l)
    ao = _dot_t(jnp.concatenate(outs, axis=1), ow_ref[:]) + ob_ref[:]
    h_ = loc_ref[:] + ao
    mu = jnp.mean(h_, axis=1, keepdims=True)
    var = jnp.mean((h_ - mu) ** 2, axis=1, keepdims=True)
    acts_ref[:] = g_ref[:] * (h_ - mu) / jnp.sqrt(var + 1e-5) + b_ref[:]


def _pa_body(acts_ref, pw_ref, w_ref, pa_ref, ps_ref):
    i = pl.program_id(0)
    pa = _gelu(_dot_t(acts_ref[:], pw_ref[:] * w_ref[:]))
    pa_ref[:] = pa
    part = jnp.sum(pa, axis=0, keepdims=True)

    @pl.when(i == 0)
    def _():
        ps_ref[:] = part

    @pl.when(i != 0)
    def _():
        ps_ref[:] = ps_ref[:] + part


def _route2_body(ps_ref, m_ref):
    ps = ps_ref[:] * (1.0 / S)
    fb = jnp.broadcast_to(ps, (NP, NP))
    fa = fb.T
    il = jax.lax.broadcasted_iota(jnp.int32, (NP, NP), 0)
    jl = jax.lax.broadcasted_iota(jnp.int32, (NP, NP), 1)
    beats = (fa > fb) | ((fa == fb) & (il < jl))
    rank = jnp.sum(beats.astype(jnp.float32), axis=0, keepdims=True)
    m_ref[:] = (rank < float(KPR)).astype(jnp.float32)


def _out_body(pa_ref, m_ref, po_ref, o_ref):
    o_ref[:] = jnp.dot(pa_ref[:] * m_ref[:], po_ref[:],
                       preferred_element_type=jnp.float32)


def kernel(x, gr_in_w, gr_in_b, gr_out_w, gr_out_b, cn_w, cn_b, patterns,
           nn_in_w, nn_in_b, nn_out_w, nn_out_b, ln_g, ln_b, pw, po,
           k_input, k_process):
    f32 = jnp.float32
    x2 = x.reshape(S, D)

    q, k, v = pl.pallas_call(
        _qkv_body,
        grid=(S // BR,),
        in_specs=[
            pl.BlockSpec((BR, D), lambda i: (i, 0)),
            pl.BlockSpec((3 * D, D), lambda i: (0, 0)),
            pl.BlockSpec((1, 3 * D), lambda i: (0, 0)),
        ],
        out_specs=[pl.BlockSpec((BR, D), lambda i: (i, 0))] * 3,
        out_shape=[jax.ShapeDtypeStruct((S, D), f32)] * 3,
    )(x2, gr_in_w, gr_in_b.reshape(1, 3 * D))

    attn_out = pl.pallas_call(
        _attn_body,
        grid=(NH // 2, S // BQ),
        in_specs=[
            pl.BlockSpec((BQ, 2 * HD), lambda p, i: (i, p)),
            pl.BlockSpec((S, 2 * HD), lambda p, i: (0, p)),
            pl.BlockSpec((S, 2 * HD), lambda p, i: (0, p)),
        ],
        out_specs=pl.BlockSpec((BQ, 2 * HD), lambda p, i: (i, p)),
        out_shape=jax.ShapeDtypeStruct((S, D), f32),
    )(q, k, v)

    local, qn, kn, vn, nsum, nmax = pl.pallas_call(
        _post_body,
        grid=(S // BR,),
        in_specs=[
            pl.BlockSpec((BR, D), lambda i: (i, 0)),
            pl.BlockSpec((D, D), lambda i: (0, 0)),
            pl.BlockSpec((1, D), lambda i: (0, 0)),
            pl.BlockSpec((NI, D), lambda i: (0, 0)),
            pl.BlockSpec((1, NI), lambda i: (0, 0)),
            pl.BlockSpec((NI, D), lambda i: (0, 0)),
            pl.BlockSpec((3 * NI, NI), lambda i: (0, 0)),
            pl.BlockSpec((1, 3 * NI), lambda i: (0, 0)),
        ],
        out_specs=[
            pl.BlockSpec((BR, NI), lambda i: (i, 0)),
            pl.BlockSpec((BR, NI), lambda i: (i, 0)),
            pl.BlockSpec((BR, NI), lambda i: (i, 0)),
            pl.BlockSpec((BR, NI), lambda i: (i, 0)),
            pl.BlockSpec((1, NI), lambda i: (0, 0)),
            pl.BlockSpec((1, NI), lambda i: (0, 0)),
        ],
        out_shape=[
            jax.ShapeDtypeStruct((S, NI), f32),
            jax.ShapeDtypeStruct((S, NI), f32),
            jax.ShapeDtypeStruct((S, NI), f32),
            jax.ShapeDtypeStruct((S, NI), f32),
            jax.ShapeDtypeStruct((1, NI), f32),
            jax.ShapeDtypeStruct((1, NI), f32),
        ],
    )(attn_out, gr_out_w, gr_out_b.reshape(1, D), cn_w, cn_b.reshape(1, NI),
      patterns, nn_in_w, nn_in_b.reshape(1, 3 * NI))

    w = pl.pallas_call(
        _route1_body,
        in_specs=[pl.BlockSpec((1, NI), lambda: (0, 0)),
                  pl.BlockSpec((1, NI), lambda: (0, 0))],
        out_specs=pl.BlockSpec((1, NI), lambda: (0, 0)),
        out_shape=jax.ShapeDtypeStruct((1, NI), f32),
    )(nsum, nmax)

    acts = pl.pallas_call(
        _nattn_body,
        grid=(S // BQ,),
        in_specs=[
            pl.BlockSpec((BQ, NI), lambda i: (i, 0)),
            pl.BlockSpec((S, NI), lambda i: (0, 0)),
            pl.BlockSpec((S, NI), lambda i: (0, 0)),
            pl.BlockSpec((BQ, NI), lambda i: (i, 0)),
            pl.BlockSpec((1, NI), lambda i: (0, 0)),
            pl.BlockSpec((1, NI), lambda i: (0, 0)),
            pl.BlockSpec((NI, NI), lambda i: (0, 0)),
            pl.BlockSpec((1, NI), lambda i: (0, 0)),
        ],
        out_specs=pl.BlockSpec((BQ, NI), lambda i: (i, 0)),
        out_shape=jax.ShapeDtypeStruct((S, NI), f32),
    )(qn, kn, vn, local, ln_g.reshape(1, NI), ln_b.reshape(1, NI),
      nn_out_w, nn_out_b.reshape(1, NI))

    pa, ps = pl.pallas_call(
        _pa_body,
        grid=(S // BR,),
        in_specs=[
            pl.BlockSpec((BR, NI), lambda i: (i, 0)),
            pl.BlockSpec((NP, NI), lambda i: (0, 0)),
            pl.BlockSpec((1, NI), lambda i: (0, 0)),
        ],
        out_specs=[
            pl.BlockSpec((BR, NP), lambda i: (i, 0)),
            pl.BlockSpec((1, NP), lambda i: (0, 0)),
        ],
        out_shape=[
            jax.ShapeDtypeStruct((S, NP), f32),
            jax.ShapeDtypeStruct((1, NP), f32),
        ],
    )(acts, pw, w)

    mask2 = pl.pallas_call(
        _route2_body,
        in_specs=[pl.BlockSpec((1, NP), lambda: (0, 0))],
        out_specs=pl.BlockSpec((1, NP), lambda: (0, 0)),
        out_shape=jax.ShapeDtypeStruct((1, NP), f32),
    )(ps)

    out = pl.pallas_call(
        _out_body,
        grid=(S // BR,),
        in_specs=[
            pl.BlockSpec((BR, NP), lambda i: (i, 0)),
            pl.BlockSpec((1, NP), lambda i: (0, 0)),
            pl.BlockSpec((NP, D), lambda i: (0, 0)),
        ],
        out_specs=pl.BlockSpec((BR, D), lambda i: (i, 0)),
        out_shape=jax.ShapeDtypeStruct((S, D), f32),
    )(pa, mask2, po)

    return out.reshape(1, S, D)


# speedup vs baseline: 3.3488x; 1.0314x over previous
import math

import jax
import jax.numpy as jnp
from jax.experimental import pallas as pl

S = 2048
D = 1024
NI = 64
NP = 128
NH = 16
HD = D // NH
NNH = 4
NHD = NI // NNH
KIN = 32
KPR = 64

BQ = 1024
BR = 256


def _gelu(x):
    return 0.5 * x * (1.0 + jax.lax.erf(x * (1.0 / math.sqrt(2.0))))


def _dot_t(a, b):
    return jax.lax.dot_general(a, b, (((1,), (1,)), ((), ())),
                               preferred_element_type=jnp.float32)


def _qkv_body(x_ref, w_ref, b_ref, q_ref, k_ref, v_ref):
    y = _dot_t(x_ref[:], w_ref[:]) + b_ref[:]
    q_ref[:] = y[:, :D]
    k_ref[:] = y[:, D:2 * D]
    v_ref[:] = y[:, 2 * D:]


def _attn_body(q_ref, k_ref, v_ref, o_ref):
    q = q_ref[:] * (1.0 / math.sqrt(HD))
    for h in (0, 1):
        sl = slice(h * HD, (h + 1) * HD)
        s = _dot_t(q[:, sl], k_ref[:, sl])
        m = jnp.max(s, axis=1, keepdims=True)
        p = jnp.exp(s - m)
        l = jnp.sum(p, axis=1, keepdims=True)
        o_ref[:, sl] = jnp.dot(p, v_ref[:, sl],
                               preferred_element_type=jnp.float32) / l


def _post_body(a_ref, wo_ref, bo_ref, cw_ref, cb_ref, pt_ref, nw_ref, nb_ref,
               loc_ref, qn_ref, kn_ref, vn_ref, nsum_ref, nmax_ref):
    i = pl.program_id(0)
    att = _dot_t(a_ref[:], wo_ref[:]) + bo_ref[:]
    na = _dot_t(att, cw_ref[:]) + cb_ref[:]
    loc = _gelu(_dot_t(att, pt_ref[:]))
    loc_ref[:] = loc
    qkvn = _dot_t(loc, nw_ref[:]) + nb_ref[:]
    qn_ref[:] = qkvn[:, :NI]
    kn_ref[:] = qkvn[:, NI:2 * NI]
    vn_ref[:] = qkvn[:, 2 * NI:]
    psum = jnp.sum(na, axis=0, keepdims=True)
    pmax = jnp.max(na, axis=0, keepdims=True)

    @pl.when(i == 0)
    def _():
        nsum_ref[:] = psum
        nmax_ref[:] = pmax

    @pl.when(i != 0)
    def _():
        nsum_ref[:] = nsum_ref[:] + psum
        nmax_ref[:] = jnp.maximum(nmax_ref[:], pmax)


def _route1_body(ns_ref, nm_ref, w_ref):
    mn = ns_ref[:] * (1.0 / S)
    mx = nm_ref[:]
    fs = 0.5 * mn + 0.3 * mx + 0.2 * mn
    p = fs - jnp.max(fs, axis=1, keepdims=True)
    e = jnp.exp(p)
    probs = e / jnp.sum(e, axis=1, keepdims=True)
    fb = jnp.broadcast_to(fs, (NI, NI))
    fa = fb.T
    il = jax.lax.broadcasted_iota(jnp.int32, (NI, NI), 0)
    jl = jax.lax.broadcasted_iota(jnp.int32, (NI, NI), 1)
    beats = (fa > fb) | ((fa == fb) & (il < jl))
    rank = jnp.sum(beats.astype(jnp.float32), axis=0, keepdims=True)
    sel = rank < float(KIN)
    w_ref[:] = jnp.where(sel, (1.0 - probs) + probs, 0.0)


def _nattn_body(qn_ref, kn_ref, vn_ref, loc_ref, g_ref, b_ref, ow_ref, ob_ref,
                acts_ref):
    qn = qn_ref[:] * (1.0 / math.sqrt(NHD))
    kn = kn_ref[:]
    vn = vn_ref[:]
    outs = []
    for h in range(NNH):
        sl = slice(h * NHD, (h + 1) * NHD)
        s = _dot_t(qn[:, sl], kn[:, sl])
        m = jnp.max(s, axis=1, keepdims=True)
        p = jnp.exp(s - m)
        l = jnp.sum(p, axis=1, keepdims=True)
        outs.append(jnp.dot(p, vn[:, sl],
                            preferred_element_type=jnp.float32) / l)
    ao = _dot_t(jnp.concatenate(outs, axis=1), ow_ref[:]) + ob_ref[:]
    h_ = loc_ref[:] + ao
    mu = jnp.mean(h_, axis=1, keepdims=True)
    var = jnp.mean((h_ - mu) ** 2, axis=1, keepdims=True)
    acts_ref[:] = g_ref[:] * (h_ - mu) / jnp.sqrt(var + 1e-5) + b_ref[:]


def _pa_body(acts_ref, pw_ref, w_ref, pa_ref, ps_ref):
    i = pl.program_id(0)
    pa = _gelu(_dot_t(acts_ref[:], pw_ref[:] * w_ref[:]))
    pa_ref[:] = pa
    part = jnp.sum(pa, axis=0, keepdims=True)

    @pl.when(i == 0)
    def _():
        ps_ref[:] = part

    @pl.when(i != 0)
    def _():
        ps_ref[:] = ps_ref[:] + part


def _route2_body(ps_ref, m_ref):
    ps = ps_ref[:] * (1.0 / S)
    fb = jnp.broadcast_to(ps, (NP, NP))
    fa = fb.T
    il = jax.lax.broadcasted_iota(jnp.int32, (NP, NP), 0)
    jl = jax.lax.broadcasted_iota(jnp.int32, (NP, NP), 1)
    beats = (fa > fb) | ((fa == fb) & (il < jl))
    rank = jnp.sum(beats.astype(jnp.float32), axis=0, keepdims=True)
    m_ref[:] = (rank < float(KPR)).astype(jnp.float32)


def _out_body(pa_ref, m_ref, po_ref, o_ref):
    o_ref[:] = jnp.dot(pa_ref[:] * m_ref[:], po_ref[:],
                       preferred_element_type=jnp.float32)


def kernel(x, gr_in_w, gr_in_b, gr_out_w, gr_out_b, cn_w, cn_b, patterns,
           nn_in_w, nn_in_b, nn_out_w, nn_out_b, ln_g, ln_b, pw, po,
           k_input, k_process):
    f32 = jnp.float32
    x2 = x.reshape(S, D)

    q, k, v = pl.pallas_call(
        _qkv_body,
        grid=(S // BR,),
        in_specs=[
            pl.BlockSpec((BR, D), lambda i: (i, 0)),
            pl.BlockSpec((3 * D, D), lambda i: (0, 0)),
            pl.BlockSpec((1, 3 * D), lambda i: (0, 0)),
        ],
        out_specs=[pl.BlockSpec((BR, D), lambda i: (i, 0))] * 3,
        out_shape=[jax.ShapeDtypeStruct((S, D), f32)] * 3,
    )(x2, gr_in_w, gr_in_b.reshape(1, 3 * D))

    attn_out = pl.pallas_call(
        _attn_body,
        grid=(NH // 2, S // BQ),
        in_specs=[
            pl.BlockSpec((BQ, 2 * HD), lambda p, i: (i, p)),
            pl.BlockSpec((S, 2 * HD), lambda p, i: (0, p)),
            pl.BlockSpec((S, 2 * HD), lambda p, i: (0, p)),
        ],
        out_specs=pl.BlockSpec((BQ, 2 * HD), lambda p, i: (i, p)),
        out_shape=jax.ShapeDtypeStruct((S, D), f32),
    )(q, k, v)

    local, qn, kn, vn, nsum, nmax = pl.pallas_call(
        _post_body,
        grid=(S // BR,),
        in_specs=[
            pl.BlockSpec((BR, D), lambda i: (i, 0)),
            pl.BlockSpec((D, D), lambda i: (0, 0)),
            pl.BlockSpec((1, D), lambda i: (0, 0)),
            pl.BlockSpec((NI, D), lambda i: (0, 0)),
            pl.BlockSpec((1, NI), lambda i: (0, 0)),
            pl.BlockSpec((NI, D), lambda i: (0, 0)),
            pl.BlockSpec((3 * NI, NI), lambda i: (0, 0)),
            pl.BlockSpec((1, 3 * NI), lambda i: (0, 0)),
        ],
        out_specs=[
            pl.BlockSpec((BR, NI), lambda i: (i, 0)),
            pl.BlockSpec((BR, NI), lambda i: (i, 0)),
            pl.BlockSpec((BR, NI), lambda i: (i, 0)),
            pl.BlockSpec((BR, NI), lambda i: (i, 0)),
            pl.BlockSpec((1, NI), lambda i: (0, 0)),
            pl.BlockSpec((1, NI), lambda i: (0, 0)),
        ],
        out_shape=[
            jax.ShapeDtypeStruct((S, NI), f32),
            jax.ShapeDtypeStruct((S, NI), f32),
            jax.ShapeDtypeStruct((S, NI), f32),
            jax.ShapeDtypeStruct((S, NI), f32),
            jax.ShapeDtypeStruct((1, NI), f32),
            jax.ShapeDtypeStruct((1, NI), f32),
        ],
    )(attn_out, gr_out_w, gr_out_b.reshape(1, D), cn_w, cn_b.reshape(1, NI),
      patterns, nn_in_w, nn_in_b.reshape(1, 3 * NI))

    w = pl.pallas_call(
        _route1_body,
        in_specs=[pl.BlockSpec((1, NI), lambda: (0, 0)),
                  pl.BlockSpec((1, NI), lambda: (0, 0))],
        out_specs=pl.BlockSpec((1, NI), lambda: (0, 0)),
        out_shape=jax.ShapeDtypeStruct((1, NI), f32),
    )(nsum, nmax)

    acts = pl.pallas_call(
        _nattn_body,
        grid=(S // BQ,),
        in_specs=[
            pl.BlockSpec((BQ, NI), lambda i: (i, 0)),
            pl.BlockSpec((S, NI), lambda i: (0, 0)),
            pl.BlockSpec((S, NI), lambda i: (0, 0)),
            pl.BlockSpec((BQ, NI), lambda i: (i, 0)),
            pl.BlockSpec((1, NI), lambda i: (0, 0)),
            pl.BlockSpec((1, NI), lambda i: (0, 0)),
            pl.BlockSpec((NI, NI), lambda i: (0, 0)),
            pl.BlockSpec((1, NI), lambda i: (0, 0)),
        ],
        out_specs=pl.BlockSpec((BQ, NI), lambda i: (i, 0)),
        out_shape=jax.ShapeDtypeStruct((S, NI), f32),
    )(qn, kn, vn, local, ln_g.reshape(1, NI), ln_b.reshape(1, NI),
      nn_out_w, nn_out_b.reshape(1, NI))

    pa, ps = pl.pallas_call(
        _pa_body,
        grid=(S // BR,),
        in_specs=[
            pl.BlockSpec((BR, NI), lambda i: (i, 0)),
            pl.BlockSpec((NP, NI), lambda i: (0, 0)),
            pl.BlockSpec((1, NI), lambda i: (0, 0)),
        ],
        out_specs=[
            pl.BlockSpec((BR, NP), lambda i: (i, 0)),
            pl.BlockSpec((1, NP), lambda i: (0, 0)),
        ],
        out_shape=[
            jax.ShapeDtypeStruct((S, NP), f32),
            jax.ShapeDtypeStruct((1, NP), f32),
        ],
    )(acts, pw, w)

    mask2 = pl.pallas_call(
        _route2_body,
        in_specs=[pl.BlockSpec((1, NP), lambda: (0, 0))],
        out_specs=pl.BlockSpec((1, NP), lambda: (0, 0)),
        out_shape=jax.ShapeDtypeStruct((1, NP), f32),
    )(ps)

    out = pl.pallas_call(
        _out_body,
        grid=(S // BR,),
        in_specs=[
            pl.BlockSpec((BR, NP), lambda i: (i, 0)),
            pl.BlockSpec((1, NP), lambda i: (0, 0)),
            pl.BlockSpec((NP, D), lambda i: (0, 0)),
        ],
        out_specs=pl.BlockSpec((BR, D), lambda i: (i, 0)),
        out_shape=jax.ShapeDtypeStruct((S, D), f32),
    )(pa, mask2, po)

    return out.reshape(1, S, D)


# speedup vs baseline: 3.7500x; 1.1198x over previous
import math

import jax
import jax.numpy as jnp
from jax.experimental import pallas as pl

S = 2048
D = 1024
NI = 64
NP = 128
NH = 16
HD = D // NH
NNH = 4
NHD = NI // NNH
KIN = 32
KPR = 64

BQ = 1024
BR = 256


def _gelu(x):
    return 0.5 * x * (1.0 + jax.lax.erf(x * (1.0 / math.sqrt(2.0))))


def _dot_t(a, b):
    return jax.lax.dot_general(a, b, (((1,), (1,)), ((), ())),
                               preferred_element_type=jnp.float32)


def _qkv_body(x_ref, w_ref, b_ref, q_ref, k_ref, v_ref):
    y = _dot_t(x_ref[:], w_ref[:]) + b_ref[:]
    q_ref[:] = y[:, :D]
    k_ref[:] = y[:, D:2 * D]
    v_ref[:] = y[:, 2 * D:]


HPS = 4


def _attn_body(q_ref, k_ref, v_ref, o_ref):
    q = q_ref[:] * (1.0 / math.sqrt(HD))
    for h in range(HPS):
        sl = slice(h * HD, (h + 1) * HD)
        s = _dot_t(q[:, sl], k_ref[:, sl])
        m = jnp.max(s, axis=1, keepdims=True)
        p = jnp.exp(s - m)
        l = jnp.sum(p, axis=1, keepdims=True)
        o_ref[:, sl] = jnp.dot(p, v_ref[:, sl],
                               preferred_element_type=jnp.float32) / l


def _post_body(a_ref, wo_ref, bo_ref, cw_ref, cb_ref, pt_ref, nw_ref, nb_ref,
               loc_ref, qn_ref, kn_ref, vn_ref, nsum_ref, nmax_ref):
    i = pl.program_id(0)
    att = _dot_t(a_ref[:], wo_ref[:]) + bo_ref[:]
    na = _dot_t(att, cw_ref[:]) + cb_ref[:]
    loc = _gelu(_dot_t(att, pt_ref[:]))
    loc_ref[:] = loc
    qkvn = _dot_t(loc, nw_ref[:]) + nb_ref[:]
    qn_ref[:] = qkvn[:, :NI]
    kn_ref[:] = qkvn[:, NI:2 * NI]
    vn_ref[:] = qkvn[:, 2 * NI:]
    psum = jnp.sum(na, axis=0, keepdims=True)
    pmax = jnp.max(na, axis=0, keepdims=True)

    @pl.when(i == 0)
    def _():
        nsum_ref[:] = psum
        nmax_ref[:] = pmax

    @pl.when(i != 0)
    def _():
        nsum_ref[:] = nsum_ref[:] + psum
        nmax_ref[:] = jnp.maximum(nmax_ref[:], pmax)


def _route1_body(ns_ref, nm_ref, w_ref):
    mn = ns_ref[:] * (1.0 / S)
    mx = nm_ref[:]
    fs = 0.5 * mn + 0.3 * mx + 0.2 * mn
    p = fs - jnp.max(fs, axis=1, keepdims=True)
    e = jnp.exp(p)
    probs = e / jnp.sum(e, axis=1, keepdims=True)
    fb = jnp.broadcast_to(fs, (NI, NI))
    fa = fb.T
    il = jax.lax.broadcasted_iota(jnp.int32, (NI, NI), 0)
    jl = jax.lax.broadcasted_iota(jnp.int32, (NI, NI), 1)
    beats = (fa > fb) | ((fa == fb) & (il < jl))
    rank = jnp.sum(beats.astype(jnp.float32), axis=0, keepdims=True)
    sel = rank < float(KIN)
    w_ref[:] = jnp.where(sel, (1.0 - probs) + probs, 0.0)


def _nattn_body(qn_ref, kn_ref, vn_ref, loc_ref, g_ref, b_ref, ow_ref, ob_ref,
                acts_ref):
    qn = qn_ref[:] * (1.0 / math.sqrt(NHD))
    kn = kn_ref[:]
    vn = vn_ref[:]
    outs = []
    for h in range(NNH):
        sl = slice(h * NHD, (h + 1) * NHD)
        s = _dot_t(qn[:, sl], kn[:, sl])
        m = jnp.max(s, axis=1, keepdims=True)
        p = jnp.exp(s - m)
        l = jnp.sum(p, axis=1, keepdims=True)
        outs.append(jnp.dot(p, vn[:, sl],
                            preferred_element_type=jnp.float32) / l)
    ao = _dot_t(jnp.concatenate(outs, axis=1), ow_ref[:]) + ob_ref[:]
    h_ = loc_ref[:] + ao
    mu = jnp.mean(h_, axis=1, keepdims=True)
    var = jnp.mean((h_ - mu) ** 2, axis=1, keepdims=True)
    acts_ref[:] = g_ref[:] * (h_ - mu) / jnp.sqrt(var + 1e-5) + b_ref[:]


def _pa_body(acts_ref, pw_ref, w_ref, pa_ref, ps_ref):
    i = pl.program_id(0)
    pa = _gelu(_dot_t(acts_ref[:], pw_ref[:] * w_ref[:]))
    pa_ref[:] = pa
    part = jnp.sum(pa, axis=0, keepdims=True)

    @pl.when(i == 0)
    def _():
        ps_ref[:] = part

    @pl.when(i != 0)
    def _():
        ps_ref[:] = ps_ref[:] + part


def _route2_body(ps_ref, m_ref):
    ps = ps_ref[:] * (1.0 / S)
    fb = jnp.broadcast_to(ps, (NP, NP))
    fa = fb.T
    il = jax.lax.broadcasted_iota(jnp.int32, (NP, NP), 0)
    jl = jax.lax.broadcasted_iota(jnp.int32, (NP, NP), 1)
    beats = (fa > fb) | ((fa == fb) & (il < jl))
    rank = jnp.sum(beats.astype(jnp.float32), axis=0, keepdims=True)
    m_ref[:] = (rank < float(KPR)).astype(jnp.float32)


def _out_body(pa_ref, m_ref, po_ref, o_ref):
    o_ref[:] = jnp.dot(pa_ref[:] * m_ref[:], po_ref[:],
                       preferred_element_type=jnp.float32)


def kernel(x, gr_in_w, gr_in_b, gr_out_w, gr_out_b, cn_w, cn_b, patterns,
           nn_in_w, nn_in_b, nn_out_w, nn_out_b, ln_g, ln_b, pw, po,
           k_input, k_process):
    f32 = jnp.float32
    x2 = x.reshape(S, D)

    q, k, v = pl.pallas_call(
        _qkv_body,
        grid=(S // BR,),
        in_specs=[
            pl.BlockSpec((BR, D), lambda i: (i, 0)),
            pl.BlockSpec((3 * D, D), lambda i: (0, 0)),
            pl.BlockSpec((1, 3 * D), lambda i: (0, 0)),
        ],
        out_specs=[pl.BlockSpec((BR, D), lambda i: (i, 0))] * 3,
        out_shape=[jax.ShapeDtypeStruct((S, D), f32)] * 3,
    )(x2, gr_in_w, gr_in_b.reshape(1, 3 * D))

    attn_out = pl.pallas_call(
        _attn_body,
        grid=(NH // HPS, S // BQ),
        in_specs=[
            pl.BlockSpec((BQ, HPS * HD), lambda p, i: (i, p)),
            pl.BlockSpec((S, HPS * HD), lambda p, i: (0, p)),
            pl.BlockSpec((S, HPS * HD), lambda p, i: (0, p)),
        ],
        out_specs=pl.BlockSpec((BQ, HPS * HD), lambda p, i: (i, p)),
        out_shape=jax.ShapeDtypeStruct((S, D), f32),
    )(q, k, v)

    local, qn, kn, vn, nsum, nmax = pl.pallas_call(
        _post_body,
        grid=(S // BR,),
        in_specs=[
            pl.BlockSpec((BR, D), lambda i: (i, 0)),
            pl.BlockSpec((D, D), lambda i: (0, 0)),
            pl.BlockSpec((1, D), lambda i: (0, 0)),
            pl.BlockSpec((NI, D), lambda i: (0, 0)),
            pl.BlockSpec((1, NI), lambda i: (0, 0)),
            pl.BlockSpec((NI, D), lambda i: (0, 0)),
            pl.BlockSpec((3 * NI, NI), lambda i: (0, 0)),
            pl.BlockSpec((1, 3 * NI), lambda i: (0, 0)),
        ],
        out_specs=[
            pl.BlockSpec((BR, NI), lambda i: (i, 0)),
            pl.BlockSpec((BR, NI), lambda i: (i, 0)),
            pl.BlockSpec((BR, NI), lambda i: (i, 0)),
            pl.BlockSpec((BR, NI), lambda i: (i, 0)),
            pl.BlockSpec((1, NI), lambda i: (0, 0)),
            pl.BlockSpec((1, NI), lambda i: (0, 0)),
        ],
        out_shape=[
            jax.ShapeDtypeStruct((S, NI), f32),
            jax.ShapeDtypeStruct((S, NI), f32),
            jax.ShapeDtypeStruct((S, NI), f32),
            jax.ShapeDtypeStruct((S, NI), f32),
            jax.ShapeDtypeStruct((1, NI), f32),
            jax.ShapeDtypeStruct((1, NI), f32),
        ],
    )(attn_out, gr_out_w, gr_out_b.reshape(1, D), cn_w, cn_b.reshape(1, NI),
      patterns, nn_in_w, nn_in_b.reshape(1, 3 * NI))

    w = pl.pallas_call(
        _route1_body,
        in_specs=[pl.BlockSpec((1, NI), lambda: (0, 0)),
                  pl.BlockSpec((1, NI), lambda: (0, 0))],
        out_specs=pl.BlockSpec((1, NI), lambda: (0, 0)),
        out_shape=jax.ShapeDtypeStruct((1, NI), f32),
    )(nsum, nmax)

    acts = pl.pallas_call(
        _nattn_body,
        grid=(S // BQ,),
        in_specs=[
            pl.BlockSpec((BQ, NI), lambda i: (i, 0)),
            pl.BlockSpec((S, NI), lambda i: (0, 0)),
            pl.BlockSpec((S, NI), lambda i: (0, 0)),
            pl.BlockSpec((BQ, NI), lambda i: (i, 0)),
            pl.BlockSpec((1, NI), lambda i: (0, 0)),
            pl.BlockSpec((1, NI), lambda i: (0, 0)),
            pl.BlockSpec((NI, NI), lambda i: (0, 0)),
            pl.BlockSpec((1, NI), lambda i: (0, 0)),
        ],
        out_specs=pl.BlockSpec((BQ, NI), lambda i: (i, 0)),
        out_shape=jax.ShapeDtypeStruct((S, NI), f32),
    )(qn, kn, vn, local, ln_g.reshape(1, NI), ln_b.reshape(1, NI),
      nn_out_w, nn_out_b.reshape(1, NI))

    pa, ps = pl.pallas_call(
        _pa_body,
        grid=(S // BR,),
        in_specs=[
            pl.BlockSpec((BR, NI), lambda i: (i, 0)),
            pl.BlockSpec((NP, NI), lambda i: (0, 0)),
            pl.BlockSpec((1, NI), lambda i: (0, 0)),
        ],
        out_specs=[
            pl.BlockSpec((BR, NP), lambda i: (i, 0)),
            pl.BlockSpec((1, NP), lambda i: (0, 0)),
        ],
        out_shape=[
            jax.ShapeDtypeStruct((S, NP), f32),
            jax.ShapeDtypeStruct((1, NP), f32),
        ],
    )(acts, pw, w)

    mask2 = pl.pallas_call(
        _route2_body,
        in_specs=[pl.BlockSpec((1, NP), lambda: (0, 0))],
        out_specs=pl.BlockSpec((1, NP), lambda: (0, 0)),
        out_shape=jax.ShapeDtypeStruct((1, NP), f32),
    )(ps)

    out = pl.pallas_call(
        _out_body,
        grid=(S // BR,),
        in_specs=[
            pl.BlockSpec((BR, NP), lambda i: (i, 0)),
            pl.BlockSpec((1, NP), lambda i: (0, 0)),
            pl.BlockSpec((NP, D), lambda i: (0, 0)),
        ],
        out_specs=pl.BlockSpec((BR, D), lambda i: (i, 0)),
        out_shape=jax.ShapeDtypeStruct((S, D), f32),
    )(pa, mask2, po)

    return out.reshape(1, S, D)


# speedup vs baseline: 3.8512x; 1.0270x over previous
import math

import jax
import jax.numpy as jnp
from jax.experimental import pallas as pl

S = 2048
D = 1024
NI = 64
NP = 128
NH = 16
HD = D // NH
NNH = 4
NHD = NI // NNH
KIN = 32
KPR = 64

BQ = 512
BR = 256


def _gelu(x):
    return 0.5 * x * (1.0 + jax.lax.erf(x * (1.0 / math.sqrt(2.0))))


def _dot_t(a, b):
    return jax.lax.dot_general(a, b, (((1,), (1,)), ((), ())),
                               preferred_element_type=jnp.float32)


def _qkv_body(x_ref, w_ref, b_ref, q_ref, k_ref, v_ref):
    y = _dot_t(x_ref[:], w_ref[:]) + b_ref[:]
    q_ref[:] = y[:, :D]
    k_ref[:] = y[:, D:2 * D]
    v_ref[:] = y[:, 2 * D:]


HPS = 8


def _attn_body(q_ref, k_ref, v_ref, o_ref):
    q = q_ref[:] * (1.0 / math.sqrt(HD))
    for h in range(HPS):
        sl = slice(h * HD, (h + 1) * HD)
        s = _dot_t(q[:, sl], k_ref[:, sl])
        m = jnp.max(s, axis=1, keepdims=True)
        p = jnp.exp(s - m)
        l = jnp.sum(p, axis=1, keepdims=True)
        o_ref[:, sl] = jnp.dot(p, v_ref[:, sl],
                               preferred_element_type=jnp.float32) / l


def _post_body(a_ref, wo_ref, bo_ref, cw_ref, cb_ref, pt_ref, nw_ref, nb_ref,
               loc_ref, qn_ref, kn_ref, vn_ref, nsum_ref, nmax_ref):
    i = pl.program_id(0)
    att = _dot_t(a_ref[:], wo_ref[:]) + bo_ref[:]
    na = _dot_t(att, cw_ref[:]) + cb_ref[:]
    loc = _gelu(_dot_t(att, pt_ref[:]))
    loc_ref[:] = loc
    qkvn = _dot_t(loc, nw_ref[:]) + nb_ref[:]
    qn_ref[:] = qkvn[:, :NI]
    kn_ref[:] = qkvn[:, NI:2 * NI]
    vn_ref[:] = qkvn[:, 2 * NI:]
    psum = jnp.sum(na, axis=0, keepdims=True)
    pmax = jnp.max(na, axis=0, keepdims=True)

    @pl.when(i == 0)
    def _():
        nsum_ref[:] = psum
        nmax_ref[:] = pmax

    @pl.when(i != 0)
    def _():
        nsum_ref[:] = nsum_ref[:] + psum
        nmax_ref[:] = jnp.maximum(nmax_ref[:], pmax)


def _route1_body(ns_ref, nm_ref, w_ref):
    mn = ns_ref[:] * (1.0 / S)
    mx = nm_ref[:]
    fs = 0.5 * mn + 0.3 * mx + 0.2 * mn
    p = fs - jnp.max(fs, axis=1, keepdims=True)
    e = jnp.exp(p)
    probs = e / jnp.sum(e, axis=1, keepdims=True)
    fb = jnp.broadcast_to(fs, (NI, NI))
    fa = fb.T
    il = jax.lax.broadcasted_iota(jnp.int32, (NI, NI), 0)
    jl = jax.lax.broadcasted_iota(jnp.int32, (NI, NI), 1)
    beats = (fa > fb) | ((fa == fb) & (il < jl))
    rank = jnp.sum(beats.astype(jnp.float32), axis=0, keepdims=True)
    sel = rank < float(KIN)
    w_ref[:] = jnp.where(sel, (1.0 - probs) + probs, 0.0)


def _nattn_body(qn_ref, kn_ref, vn_ref, loc_ref, g_ref, b_ref, ow_ref, ob_ref,
                acts_ref):
    qn = qn_ref[:] * (1.0 / math.sqrt(NHD))
    kn = kn_ref[:]
    vn = vn_ref[:]
    outs = []
    for h in range(NNH):
        sl = slice(h * NHD, (h + 1) * NHD)
        s = _dot_t(qn[:, sl], kn[:, sl])
        m = jnp.max(s, axis=1, keepdims=True)
        p = jnp.exp(s - m)
        l = jnp.sum(p, axis=1, keepdims=True)
        outs.append(jnp.dot(p, vn[:, sl],
                            preferred_element_type=jnp.float32) / l)
    ao = _dot_t(jnp.concatenate(outs, axis=1), ow_ref[:]) + ob_ref[:]
    h_ = loc_ref[:] + ao
    mu = jnp.mean(h_, axis=1, keepdims=True)
    var = jnp.mean((h_ - mu) ** 2, axis=1, keepdims=True)
    acts_ref[:] = g_ref[:] * (h_ - mu) / jnp.sqrt(var + 1e-5) + b_ref[:]


def _pa_body(acts_ref, pw_ref, w_ref, pa_ref, ps_ref):
    i = pl.program_id(0)
    pa = _gelu(_dot_t(acts_ref[:], pw_ref[:] * w_ref[:]))
    pa_ref[:] = pa
    part = jnp.sum(pa, axis=0, keepdims=True)

    @pl.when(i == 0)
    def _():
        ps_ref[:] = part

    @pl.when(i != 0)
    def _():
        ps_ref[:] = ps_ref[:] + part


def _route2_body(ps_ref, m_ref):
    ps = ps_ref[:] * (1.0 / S)
    fb = jnp.broadcast_to(ps, (NP, NP))
    fa = fb.T
    il = jax.lax.broadcasted_iota(jnp.int32, (NP, NP), 0)
    jl = jax.lax.broadcasted_iota(jnp.int32, (NP, NP), 1)
    beats = (fa > fb) | ((fa == fb) & (il < jl))
    rank = jnp.sum(beats.astype(jnp.float32), axis=0, keepdims=True)
    m_ref[:] = (rank < float(KPR)).astype(jnp.float32)


def _out_body(pa_ref, m_ref, po_ref, o_ref):
    o_ref[:] = jnp.dot(pa_ref[:] * m_ref[:], po_ref[:],
                       preferred_element_type=jnp.float32)


def kernel(x, gr_in_w, gr_in_b, gr_out_w, gr_out_b, cn_w, cn_b, patterns,
           nn_in_w, nn_in_b, nn_out_w, nn_out_b, ln_g, ln_b, pw, po,
           k_input, k_process):
    f32 = jnp.float32
    x2 = x.reshape(S, D)

    q, k, v = pl.pallas_call(
        _qkv_body,
        grid=(S // BR,),
        in_specs=[
            pl.BlockSpec((BR, D), lambda i: (i, 0)),
            pl.BlockSpec((3 * D, D), lambda i: (0, 0)),
            pl.BlockSpec((1, 3 * D), lambda i: (0, 0)),
        ],
        out_specs=[pl.BlockSpec((BR, D), lambda i: (i, 0))] * 3,
        out_shape=[jax.ShapeDtypeStruct((S, D), f32)] * 3,
    )(x2, gr_in_w, gr_in_b.reshape(1, 3 * D))

    attn_out = pl.pallas_call(
        _attn_body,
        grid=(NH // HPS, S // BQ),
        in_specs=[
            pl.BlockSpec((BQ, HPS * HD), lambda p, i: (i, p)),
            pl.BlockSpec((S, HPS * HD), lambda p, i: (0, p)),
            pl.BlockSpec((S, HPS * HD), lambda p, i: (0, p)),
        ],
        out_specs=pl.BlockSpec((BQ, HPS * HD), lambda p, i: (i, p)),
        out_shape=jax.ShapeDtypeStruct((S, D), f32),
    )(q, k, v)

    local, qn, kn, vn, nsum, nmax = pl.pallas_call(
        _post_body,
        grid=(S // BR,),
        in_specs=[
            pl.BlockSpec((BR, D), lambda i: (i, 0)),
            pl.BlockSpec((D, D), lambda i: (0, 0)),
            pl.BlockSpec((1, D), lambda i: (0, 0)),
            pl.BlockSpec((NI, D), lambda i: (0, 0)),
            pl.BlockSpec((1, NI), lambda i: (0, 0)),
            pl.BlockSpec((NI, D), lambda i: (0, 0)),
            pl.BlockSpec((3 * NI, NI), lambda i: (0, 0)),
            pl.BlockSpec((1, 3 * NI), lambda i: (0, 0)),
        ],
        out_specs=[
            pl.BlockSpec((BR, NI), lambda i: (i, 0)),
            pl.BlockSpec((BR, NI), lambda i: (i, 0)),
            pl.BlockSpec((BR, NI), lambda i: (i, 0)),
            pl.BlockSpec((BR, NI), lambda i: (i, 0)),
            pl.BlockSpec((1, NI), lambda i: (0, 0)),
            pl.BlockSpec((1, NI), lambda i: (0, 0)),
        ],
        out_shape=[
            jax.ShapeDtypeStruct((S, NI), f32),
            jax.ShapeDtypeStruct((S, NI), f32),
            jax.ShapeDtypeStruct((S, NI), f32),
            jax.ShapeDtypeStruct((S, NI), f32),
            jax.ShapeDtypeStruct((1, NI), f32),
            jax.ShapeDtypeStruct((1, NI), f32),
        ],
    )(attn_out, gr_out_w, gr_out_b.reshape(1, D), cn_w, cn_b.reshape(1, NI),
      patterns, nn_in_w, nn_in_b.reshape(1, 3 * NI))

    w = pl.pallas_call(
        _route1_body,
        in_specs=[pl.BlockSpec((1, NI), lambda: (0, 0)),
                  pl.BlockSpec((1, NI), lambda: (0, 0))],
        out_specs=pl.BlockSpec((1, NI), lambda: (0, 0)),
        out_shape=jax.ShapeDtypeStruct((1, NI), f32),
    )(nsum, nmax)

    acts = pl.pallas_call(
        _nattn_body,
        grid=(S // BQ,),
        in_specs=[
            pl.BlockSpec((BQ, NI), lambda i: (i, 0)),
            pl.BlockSpec((S, NI), lambda i: (0, 0)),
            pl.BlockSpec((S, NI), lambda i: (0, 0)),
            pl.BlockSpec((BQ, NI), lambda i: (i, 0)),
            pl.BlockSpec((1, NI), lambda i: (0, 0)),
            pl.BlockSpec((1, NI), lambda i: (0, 0)),
            pl.BlockSpec((NI, NI), lambda i: (0, 0)),
            pl.BlockSpec((1, NI), lambda i: (0, 0)),
        ],
        out_specs=pl.BlockSpec((BQ, NI), lambda i: (i, 0)),
        out_shape=jax.ShapeDtypeStruct((S, NI), f32),
    )(qn, kn, vn, local, ln_g.reshape(1, NI), ln_b.reshape(1, NI),
      nn_out_w, nn_out_b.reshape(1, NI))

    pa, ps = pl.pallas_call(
        _pa_body,
        grid=(S // BR,),
        in_specs=[
            pl.BlockSpec((BR, NI), lambda i: (i, 0)),
            pl.BlockSpec((NP, NI), lambda i: (0, 0)),
            pl.BlockSpec((1, NI), lambda i: (0, 0)),
        ],
        out_specs=[
            pl.BlockSpec((BR, NP), lambda i: (i, 0)),
            pl.BlockSpec((1, NP), lambda i: (0, 0)),
        ],
        out_shape=[
            jax.ShapeDtypeStruct((S, NP), f32),
            jax.ShapeDtypeStruct((1, NP), f32),
        ],
    )(acts, pw, w)

    mask2 = pl.pallas_call(
        _route2_body,
        in_specs=[pl.BlockSpec((1, NP), lambda: (0, 0))],
        out_specs=pl.BlockSpec((1, NP), lambda: (0, 0)),
        out_shape=jax.ShapeDtypeStruct((1, NP), f32),
    )(ps)

    out = pl.pallas_call(
        _out_body,
        grid=(S // BR,),
        in_specs=[
            pl.BlockSpec((BR, NP), lambda i: (i, 0)),
            pl.BlockSpec((1, NP), lambda i: (0, 0)),
            pl.BlockSpec((NP, D), lambda i: (0, 0)),
        ],
        out_specs=pl.BlockSpec((BR, D), lambda i: (i, 0)),
        out_shape=jax.ShapeDtypeStruct((S, D), f32),
    )(pa, mask2, po)

    return out.reshape(1, S, D)


# speedup vs baseline: 3.9939x; 1.0371x over previous
import math

import jax
import jax.numpy as jnp
from jax.experimental import pallas as pl

S = 2048
D = 1024
NI = 64
NP = 128
NH = 16
HD = D // NH
NNH = 4
NHD = NI // NNH
KIN = 32
KPR = 64

BQ = 512
BR = 256


def _gelu(x):
    return 0.5 * x * (1.0 + jax.lax.erf(x * (1.0 / math.sqrt(2.0))))


def _dot_t(a, b):
    return jax.lax.dot_general(a, b, (((1,), (1,)), ((), ())),
                               preferred_element_type=jnp.float32)


def _qkv_body(x_ref, w_ref, b_ref, q_ref, k_ref, v_ref):
    y = _dot_t(x_ref[:], w_ref[:]) + b_ref[:]
    q_ref[:] = y[:, :D]
    k_ref[:] = y[:, D:2 * D]
    v_ref[:] = y[:, 2 * D:]


HPS = 8


def _attn_body(q_ref, k_ref, v_ref, o_ref):
    q = q_ref[:] * (1.0 / math.sqrt(HD))
    for h in range(HPS):
        sl = slice(h * HD, (h + 1) * HD)
        s = _dot_t(q[:, sl], k_ref[:, sl])
        m = jnp.max(s, axis=1, keepdims=True)
        p = jnp.exp(s - m)
        l = jnp.sum(p, axis=1, keepdims=True)
        o_ref[:, sl] = jnp.dot(p, v_ref[:, sl],
                               preferred_element_type=jnp.float32) / l


def _post_body(a_ref, wo_ref, bo_ref, cw_ref, cb_ref, pt_ref, nw_ref, nb_ref,
               loc_ref, qn_ref, kn_ref, vn_ref, nsum_ref, nmax_ref):
    i = pl.program_id(0)
    att = _dot_t(a_ref[:], wo_ref[:]) + bo_ref[:]
    na = _dot_t(att, cw_ref[:]) + cb_ref[:]
    loc = _gelu(_dot_t(att, pt_ref[:]))
    loc_ref[:] = loc
    qkvn = _dot_t(loc, nw_ref[:]) + nb_ref[:]
    qn_ref[:] = qkvn[:, :NI]
    kn_ref[:] = qkvn[:, NI:2 * NI]
    vn_ref[:] = qkvn[:, 2 * NI:]
    psum = jnp.sum(na, axis=0, keepdims=True)
    pmax = jnp.max(na, axis=0, keepdims=True)

    @pl.when(i == 0)
    def _():
        nsum_ref[:] = psum
        nmax_ref[:] = pmax

    @pl.when(i != 0)
    def _():
        nsum_ref[:] = nsum_ref[:] + psum
        nmax_ref[:] = jnp.maximum(nmax_ref[:], pmax)


def _route1_body(ns_ref, nm_ref, w_ref):
    mn = ns_ref[:] * (1.0 / S)
    mx = nm_ref[:]
    fs = 0.5 * mn + 0.3 * mx + 0.2 * mn
    p = fs - jnp.max(fs, axis=1, keepdims=True)
    e = jnp.exp(p)
    probs = e / jnp.sum(e, axis=1, keepdims=True)
    fb = jnp.broadcast_to(fs, (NI, NI))
    fa = fb.T
    il = jax.lax.broadcasted_iota(jnp.int32, (NI, NI), 0)
    jl = jax.lax.broadcasted_iota(jnp.int32, (NI, NI), 1)
    beats = (fa > fb) | ((fa == fb) & (il < jl))
    rank = jnp.sum(beats.astype(jnp.float32), axis=0, keepdims=True)
    sel = rank < float(KIN)
    w_ref[:] = jnp.where(sel, (1.0 - probs) + probs, 0.0)


def _nattn_body(qn_ref, kn_ref, vn_ref, loc_ref, g_ref, b_ref, ow_ref, ob_ref,
                pw_ref, w_ref, pa_ref, ps_ref):
    i = pl.program_id(0)
    qn = qn_ref[:] * (1.0 / math.sqrt(NHD))
    kn = kn_ref[:]
    vn = vn_ref[:]
    outs = []
    for h in range(NNH):
        sl = slice(h * NHD, (h + 1) * NHD)
        s = _dot_t(qn[:, sl], kn[:, sl])
        m = jnp.max(s, axis=1, keepdims=True)
        p = jnp.exp(s - m)
        l = jnp.sum(p, axis=1, keepdims=True)
        outs.append(jnp.dot(p, vn[:, sl],
                            preferred_element_type=jnp.float32) / l)
    ao = _dot_t(jnp.concatenate(outs, axis=1), ow_ref[:]) + ob_ref[:]
    h_ = loc_ref[:] + ao
    mu = jnp.mean(h_, axis=1, keepdims=True)
    var = jnp.mean((h_ - mu) ** 2, axis=1, keepdims=True)
    acts = g_ref[:] * (h_ - mu) / jnp.sqrt(var + 1e-5) + b_ref[:]
    pa = _gelu(_dot_t(acts, pw_ref[:] * w_ref[:]))
    pa_ref[:] = pa
    part = jnp.sum(pa, axis=0, keepdims=True)

    @pl.when(i == 0)
    def _():
        ps_ref[:] = part

    @pl.when(i != 0)
    def _():
        ps_ref[:] = ps_ref[:] + part


def _route2_body(ps_ref, m_ref):
    ps = ps_ref[:] * (1.0 / S)
    fb = jnp.broadcast_to(ps, (NP, NP))
    fa = fb.T
    il = jax.lax.broadcasted_iota(jnp.int32, (NP, NP), 0)
    jl = jax.lax.broadcasted_iota(jnp.int32, (NP, NP), 1)
    beats = (fa > fb) | ((fa == fb) & (il < jl))
    rank = jnp.sum(beats.astype(jnp.float32), axis=0, keepdims=True)
    m_ref[:] = (rank < float(KPR)).astype(jnp.float32)


def _out_body(pa_ref, m_ref, po_ref, o_ref):
    o_ref[:] = jnp.dot(pa_ref[:] * m_ref[:], po_ref[:],
                       preferred_element_type=jnp.float32)


def kernel(x, gr_in_w, gr_in_b, gr_out_w, gr_out_b, cn_w, cn_b, patterns,
           nn_in_w, nn_in_b, nn_out_w, nn_out_b, ln_g, ln_b, pw, po,
           k_input, k_process):
    f32 = jnp.float32
    x2 = x.reshape(S, D)

    q, k, v = pl.pallas_call(
        _qkv_body,
        grid=(S // BR,),
        in_specs=[
            pl.BlockSpec((BR, D), lambda i: (i, 0)),
            pl.BlockSpec((3 * D, D), lambda i: (0, 0)),
            pl.BlockSpec((1, 3 * D), lambda i: (0, 0)),
        ],
        out_specs=[pl.BlockSpec((BR, D), lambda i: (i, 0))] * 3,
        out_shape=[jax.ShapeDtypeStruct((S, D), f32)] * 3,
    )(x2, gr_in_w, gr_in_b.reshape(1, 3 * D))

    attn_out = pl.pallas_call(
        _attn_body,
        grid=(NH // HPS, S // BQ),
        in_specs=[
            pl.BlockSpec((BQ, HPS * HD), lambda p, i: (i, p)),
            pl.BlockSpec((S, HPS * HD), lambda p, i: (0, p)),
            pl.BlockSpec((S, HPS * HD), lambda p, i: (0, p)),
        ],
        out_specs=pl.BlockSpec((BQ, HPS * HD), lambda p, i: (i, p)),
        out_shape=jax.ShapeDtypeStruct((S, D), f32),
    )(q, k, v)

    local, qn, kn, vn, nsum, nmax = pl.pallas_call(
        _post_body,
        grid=(S // BR,),
        in_specs=[
            pl.BlockSpec((BR, D), lambda i: (i, 0)),
            pl.BlockSpec((D, D), lambda i: (0, 0)),
            pl.BlockSpec((1, D), lambda i: (0, 0)),
            pl.BlockSpec((NI, D), lambda i: (0, 0)),
            pl.BlockSpec((1, NI), lambda i: (0, 0)),
            pl.BlockSpec((NI, D), lambda i: (0, 0)),
            pl.BlockSpec((3 * NI, NI), lambda i: (0, 0)),
            pl.BlockSpec((1, 3 * NI), lambda i: (0, 0)),
        ],
        out_specs=[
            pl.BlockSpec((BR, NI), lambda i: (i, 0)),
            pl.BlockSpec((BR, NI), lambda i: (i, 0)),
            pl.BlockSpec((BR, NI), lambda i: (i, 0)),
            pl.BlockSpec((BR, NI), lambda i: (i, 0)),
            pl.BlockSpec((1, NI), lambda i: (0, 0)),
            pl.BlockSpec((1, NI), lambda i: (0, 0)),
        ],
        out_shape=[
            jax.ShapeDtypeStruct((S, NI), f32),
            jax.ShapeDtypeStruct((S, NI), f32),
            jax.ShapeDtypeStruct((S, NI), f32),
            jax.ShapeDtypeStruct((S, NI), f32),
            jax.ShapeDtypeStruct((1, NI), f32),
            jax.ShapeDtypeStruct((1, NI), f32),
        ],
    )(attn_out, gr_out_w, gr_out_b.reshape(1, D), cn_w, cn_b.reshape(1, NI),
      patterns, nn_in_w, nn_in_b.reshape(1, 3 * NI))

    w = pl.pallas_call(
        _route1_body,
        in_specs=[pl.BlockSpec((1, NI), lambda: (0, 0)),
                  pl.BlockSpec((1, NI), lambda: (0, 0))],
        out_specs=pl.BlockSpec((1, NI), lambda: (0, 0)),
        out_shape=jax.ShapeDtypeStruct((1, NI), f32),
    )(nsum, nmax)

    pa, ps = pl.pallas_call(
        _nattn_body,
        grid=(S // BQ,),
        in_specs=[
            pl.BlockSpec((BQ, NI), lambda i: (i, 0)),
            pl.BlockSpec((S, NI), lambda i: (0, 0)),
            pl.BlockSpec((S, NI), lambda i: (0, 0)),
            pl.BlockSpec((BQ, NI), lambda i: (i, 0)),
            pl.BlockSpec((1, NI), lambda i: (0, 0)),
            pl.BlockSpec((1, NI), lambda i: (0, 0)),
            pl.BlockSpec((NI, NI), lambda i: (0, 0)),
            pl.BlockSpec((1, NI), lambda i: (0, 0)),
            pl.BlockSpec((NP, NI), lambda i: (0, 0)),
            pl.BlockSpec((1, NI), lambda i: (0, 0)),
        ],
        out_specs=[
            pl.BlockSpec((BQ, NP), lambda i: (i, 0)),
            pl.BlockSpec((1, NP), lambda i: (0, 0)),
        ],
        out_shape=[
            jax.ShapeDtypeStruct((S, NP), f32),
            jax.ShapeDtypeStruct((1, NP), f32),
        ],
    )(qn, kn, vn, local, ln_g.reshape(1, NI), ln_b.reshape(1, NI),
      nn_out_w, nn_out_b.reshape(1, NI), pw, w)

    mask2 = pl.pallas_call(
        _route2_body,
        in_specs=[pl.BlockSpec((1, NP), lambda: (0, 0))],
        out_specs=pl.BlockSpec((1, NP), lambda: (0, 0)),
        out_shape=jax.ShapeDtypeStruct((1, NP), f32),
    )(ps)

    out = pl.pallas_call(
        _out_body,
        grid=(S // BR,),
        in_specs=[
            pl.BlockSpec((BR, NP), lambda i: (i, 0)),
            pl.BlockSpec((1, NP), lambda i: (0, 0)),
            pl.BlockSpec((NP, D), lambda i: (0, 0)),
        ],
        out_specs=pl.BlockSpec((BR, D), lambda i: (i, 0)),
        out_shape=jax.ShapeDtypeStruct((S, D), f32),
    )(pa, mask2, po)

    return out.reshape(1, S, D)
